# baseline (speedup 1.0000x reference)
import jax
import jax.numpy as jnp
from jax.experimental import pallas as pl
from jax.experimental.pallas import tpu as pltpu

TAU = 1.0
BM = 1024
SUB = 256


NKC = 4


def _mm_kernel(idx_ref, x_ref, w_ref, b_ref, o_ref, wbf_ref):
    i = pl.program_id(0)
    KC = w_ref.shape[2]

    for c in range(NKC):
        @pl.when(i == c)
        def _(c=c):
            cols = pl.ds(c * KC, KC)
            wbf_ref[cols, :] = w_ref[0, 0].astype(jnp.bfloat16)
            for t in range(BM // SUB):
                rows = pl.ds(t * SUB, SUB)
                x = x_ref[rows, cols].astype(jnp.bfloat16)
                acc = jax.lax.dot_general(
                    x, wbf_ref[cols, :], (((1,), (0,)), ((), ())),
                    preferred_element_type=jnp.float32,
                )
                if c == 0:
                    o_ref[rows, :] = acc
                elif c == NKC - 1:
                    o_ref[rows, :] = jnp.tanh(acc + o_ref[rows, :] + b_ref[0, 0])
                else:
                    o_ref[rows, :] = acc + o_ref[rows, :]

    @pl.when(i >= NKC)
    def _():
        for t in range(BM // SUB):
            rows = pl.ds(t * SUB, SUB)
            x = x_ref[rows, :].astype(jnp.bfloat16)
            acc = jax.lax.dot_general(
                x, wbf_ref[...], (((1,), (0,)), ((), ())),
                preferred_element_type=jnp.float32,
            )
            o_ref[rows, :] = jnp.tanh(acc + b_ref[0, 0])


def kernel(args, input, embed, labels, bts, ctx, eda, weights, W, b):
    E, D, _ = W.shape
    Bb, S, _ = embed.shape
    M = Bb * S
    KC = D // NKC

    route_key = jax.random.fold_in(jax.random.key(0), 123)
    u = jax.random.uniform(route_key, weights.shape, minval=1e-6, maxval=1.0 - 1e-6)
    g = -jnp.log(-jnp.log(u))
    y_soft = jax.nn.softmax((weights + g) / TAU)
    idx = jnp.argmax(y_soft).astype(jnp.int32).reshape((1,))

    x2d = embed.reshape(M, D)
    b3 = b.reshape(E, 1, D)
    W4 = W.reshape(E, NKC, KC, D)

    grid_spec = pltpu.PrefetchScalarGridSpec(
        num_scalar_prefetch=1,
        grid=(M // BM + NKC - 1,),
        in_specs=[
            pl.BlockSpec((BM, D), lambda i, idx: (jnp.maximum(i - (NKC - 1), 0), 0)),
            pl.BlockSpec((1, 1, KC, D), lambda i, idx: (idx[0], jnp.minimum(i, NKC - 1), 0, 0)),
            pl.BlockSpec((1, 1, D), lambda i, idx: (idx[0], 0, 0)),
        ],
        out_specs=pl.BlockSpec((BM, D), lambda i, idx: (jnp.maximum(i - (NKC - 1), 0), 0)),
        scratch_shapes=[
            pltpu.VMEM((D, D), jnp.bfloat16),
        ],
    )
    out = pl.pallas_call(
        _mm_kernel,
        grid_spec=grid_spec,
        out_shape=jax.ShapeDtypeStruct((M, D), jnp.float32),
        compiler_params=pltpu.CompilerParams(
            vmem_limit_bytes=63 * 1024 * 1024,
        ),
    )(idx, x2d, W4, b3)
    return (input, out.reshape(Bb, S, D))


# speedup vs baseline: 1.0060x; 1.0060x over previous
import jax
import jax.numpy as jnp
from jax.experimental import pallas as pl
from jax.experimental.pallas import tpu as pltpu

TAU = 1.0
BM = 1024
SUB = 256


def _mm_kernel(idx_ref, x_ref, w_ref, b_ref, o_ref, wbf_ref):
    i = pl.program_id(0)
    K2 = w_ref.shape[2]

    @pl.when(i == 0)
    def _():
        wbf_ref[: K2, :] = w_ref[0, 0].astype(jnp.bfloat16)
        for t in range(BM // SUB):
            rows = pl.ds(t * SUB, SUB)
            x = x_ref[rows, :K2].astype(jnp.bfloat16)
            o_ref[rows, :] = jax.lax.dot_general(
                x, wbf_ref[: K2, :], (((1,), (0,)), ((), ())),
                preferred_element_type=jnp.float32,
            )

    @pl.when(i == 1)
    def _():
        wbf_ref[K2:, :] = w_ref[0, 0].astype(jnp.bfloat16)
        for t in range(BM // SUB):
            rows = pl.ds(t * SUB, SUB)
            x = x_ref[rows, K2:].astype(jnp.bfloat16)
            acc = jax.lax.dot_general(
                x, wbf_ref[K2:, :], (((1,), (0,)), ((), ())),
                preferred_element_type=jnp.float32,
            )
            o_ref[rows, :] = jnp.tanh(acc + o_ref[rows, :] + b_ref[0, 0])

    @pl.when(i >= 2)
    def _():
        for t in range(BM // SUB):
            rows = pl.ds(t * SUB, SUB)
            x = x_ref[rows, :].astype(jnp.bfloat16)
            acc = jax.lax.dot_general(
                x, wbf_ref[...], (((1,), (0,)), ((), ())),
                preferred_element_type=jnp.float32,
            )
            o_ref[rows, :] = jnp.tanh(acc + b_ref[0, 0])


def kernel(args, input, embed, labels, bts, ctx, eda, weights, W, b):
    E, D, _ = W.shape
    Bb, S, _ = embed.shape
    M = Bb * S
    K2 = D // 2

    route_key = jax.random.fold_in(jax.random.key(0), 123)
    u = jax.random.uniform(route_key, weights.shape, minval=1e-6, maxval=1.0 - 1e-6)
    g = -jnp.log(-jnp.log(u))
    y_soft = jax.nn.softmax((weights + g) / TAU)
    idx = jnp.argmax(y_soft).astype(jnp.int32).reshape((1,))

    x2d = embed.reshape(M, D)
    b3 = b.reshape(E, 1, D)
    W4 = W.reshape(E, 2, K2, D)

    grid_spec = pltpu.PrefetchScalarGridSpec(
        num_scalar_prefetch=1,
        grid=(M // BM + 1,),
        in_specs=[
            pl.BlockSpec((BM, D), lambda i, idx: (jnp.maximum(i - 1, 0), 0)),
            pl.BlockSpec((1, 1, K2, D), lambda i, idx: (idx[0], jnp.minimum(i, 1), 0, 0)),
            pl.BlockSpec((1, 1, D), lambda i, idx: (idx[0], 0, 0)),
        ],
        out_specs=pl.BlockSpec((BM, D), lambda i, idx: (jnp.maximum(i - 1, 0), 0)),
        scratch_shapes=[
            pltpu.VMEM((D, D), jnp.bfloat16),
        ],
    )
    out = pl.pallas_call(
        _mm_kernel,
        grid_spec=grid_spec,
        out_shape=jax.ShapeDtypeStruct((M, D), jnp.float32),
        compiler_params=pltpu.CompilerParams(
            vmem_limit_bytes=63 * 1024 * 1024,
        ),
    )(idx, x2d, W4, b3)
    return (input, out.reshape(Bb, S, D))
